# baseline (speedup 1.0000x reference)
import functools

import jax
import jax.numpy as jnp
from jax import lax
from jax.experimental import pallas as pl
from jax.experimental.pallas import tpu as pltpu
from jax.experimental.pallas import tpu_sc as plsc


def _make_gather_sum(vocab_rows, emb, batch, ctx):
  info = plsc.get_sparse_core_info()
  nc, ns = info.num_cores, info.num_subcores
  nw = nc * ns
  bpw = batch // nw
  ipw = bpw * ctx
  chunk = 128
  nchunk = ipw // chunk
  assert ipw % chunk == 0

  mesh = plsc.VectorSubcoreMesh(core_axis_name="c", subcore_axis_name="s")

  @functools.partial(
      pl.kernel,
      mesh=mesh,
      out_type=jax.ShapeDtypeStruct((batch, emb), jnp.float32),
      compiler_params=pltpu.CompilerParams(use_tc_tiling_on_sc=False),
      scratch_types=[
          pltpu.VMEM((nchunk, chunk), jnp.int32),
          pltpu.VMEM((ipw, emb), jnp.float32),
          pltpu.VMEM((bpw, emb), jnp.float32),
          pltpu.SemaphoreType.DMA,
      ],
  )
  def gather_sum(idx_hbm, table_hbm, out_hbm, idx_v, rows_v, acc_v, sem):
    wid = lax.axis_index("s") * nc + lax.axis_index("c")
    pltpu.sync_copy(idx_hbm.at[wid], idx_v)
    copies = []
    for j in range(nchunk):
      copies.append(
          pltpu.async_copy(
              table_hbm.at[idx_v.at[j]],
              rows_v.at[pl.ds(j * chunk, chunk)],
              sem,
          ))
    for c in copies:
      c.wait()

    def body(i, carry):
      for jj in range(emb // 16):
        sl = pl.ds(jj * 16, 16)
        acc = rows_v[i * ctx, sl]
        for c in range(1, ctx):
          acc = acc + rows_v[i * ctx + c, sl]
        acc_v[i, sl] = acc
      return carry

    lax.fori_loop(0, bpw, body, 0)
    pltpu.sync_copy(acc_v, out_hbm.at[pl.ds(wid * bpw, bpw)])

  return gather_sum


_TV = 1024


def _logits(emb_ref, w_ref, b_ref):
  acc = lax.dot_general(
      emb_ref[...], w_ref[...],
      dimension_numbers=(((1,), (1,)), ((), ())),
      preferred_element_type=jnp.float32,
  )
  return acc + b_ref[...]


def _stats_body(nv, vocab, emb_ref, w_ref, b_ref, logz_ref, m_ref, s_ref):
  v = pl.program_id(0)

  @pl.when(v == 0)
  def _():
    m_ref[...] = jnp.full_like(m_ref, -jnp.inf)
    s_ref[...] = jnp.zeros_like(s_ref)

  logits = _logits(emb_ref, w_ref, b_ref)
  col = v * _TV + lax.broadcasted_iota(jnp.int32, logits.shape, 1)
  logits = jnp.where(col < vocab, logits, -jnp.inf)
  tile_max = jnp.max(logits, axis=1, keepdims=True)
  m_old = m_ref[...]
  m_new = jnp.maximum(m_old, tile_max)
  s_ref[...] = (s_ref[...] * jnp.exp(m_old - m_new)
                + jnp.sum(jnp.exp(logits - m_new), axis=1, keepdims=True))
  m_ref[...] = m_new

  @pl.when(v == nv - 1)
  def _():
    logz_ref[...] = m_ref[...] + jnp.log(s_ref[...])


def _project_body(emb_ref, w_ref, b_ref, logz_ref, out_ref):
  out_ref[...] = _logits(emb_ref, w_ref, b_ref) - logz_ref[...]


def _log_softmax_linear(embeds, W, b2):
  batch, emb = embeds.shape
  vocab = W.shape[0]
  nv = pl.cdiv(vocab, _TV)

  logz = pl.pallas_call(
      functools.partial(_stats_body, nv, vocab),
      grid=(nv,),
      in_specs=[
          pl.BlockSpec((batch, emb), lambda v: (0, 0)),
          pl.BlockSpec((_TV, emb), lambda v: (v, 0)),
          pl.BlockSpec((1, _TV), lambda v: (0, v)),
      ],
      out_specs=pl.BlockSpec((batch, 1), lambda v: (0, 0)),
      out_shape=jax.ShapeDtypeStruct((batch, 1), jnp.float32),
      scratch_shapes=[
          pltpu.VMEM((batch, 1), jnp.float32),
          pltpu.VMEM((batch, 1), jnp.float32),
      ],
      compiler_params=pltpu.CompilerParams(
          dimension_semantics=("arbitrary",)),
  )(embeds, W, b2)

  out = pl.pallas_call(
      _project_body,
      grid=(nv,),
      in_specs=[
          pl.BlockSpec((batch, emb), lambda v: (0, 0)),
          pl.BlockSpec((_TV, emb), lambda v: (v, 0)),
          pl.BlockSpec((1, _TV), lambda v: (0, v)),
          pl.BlockSpec((batch, 1), lambda v: (0, 0)),
      ],
      out_specs=pl.BlockSpec((batch, _TV), lambda v: (0, v)),
      out_shape=jax.ShapeDtypeStruct((batch, vocab), jnp.float32),
      compiler_params=pltpu.CompilerParams(
          dimension_semantics=("arbitrary",)),
  )(embeds, W, b2, logz)
  return out


def kernel(inputs, emb_table, W, b):
  ctx, batch = inputs.shape
  vocab, emb = emb_table.shape
  idx = inputs.T.reshape(32, -1, 128)
  embeds = _make_gather_sum(vocab, emb, batch, ctx)(idx, emb_table)
  return _log_softmax_linear(embeds, W, b.reshape(1, -1))
